# Initial kernel scaffold; baseline (speedup 1.0000x reference)
import jax
import jax.numpy as jnp
from jax.experimental import pallas as pl


def kernel(h, edge_index, snorm_n, label, delta, params):
    raise NotImplementedError("write your pallas kernel here")


# speedup vs baseline: 2.2693x; 2.2693x over previous
import functools

import jax
import jax.numpy as jnp
from jax import lax
from jax.experimental import pallas as pl
from jax.experimental.pallas import tpu as pltpu
from jax.experimental.pallas import tpu_sc as plsc

NN = 10000
EE = 320000
IN_DIM = 32
HID = 128
NC_CLS = 6
D2 = 134
D2P = 144

NCORE = 2
NSUB = 16
NW = NCORE * NSUB
CHUNK = 128
CH_PER_W = -(-EE // (NW * CHUNK))
EPW = CH_PER_W * CHUNK
EPAD = EPW * NW
NPAD = 10240
RPS = NPAD // NSUB

@functools.cache
def _mesh():
    return plsc.VectorSubcoreMesh(core_axis_name="c", subcore_axis_name="s",
                                  num_cores=NCORE, num_subcores=NSUB)


def _sc_degrees(srcp, dstp, ones_blk, zeros_blk):
    @functools.partial(
        pl.kernel,
        out_type=jax.ShapeDtypeStruct((NCORE, 2, NPAD, 16), jnp.float32),
        mesh=_mesh(),
        compiler_params=pltpu.CompilerParams(use_tc_tiling_on_sc=False),
        scratch_types=[
            pltpu.VMEM((CHUNK,), jnp.int32),
            pltpu.VMEM((CHUNK,), jnp.int32),
            pltpu.VMEM((CHUNK, 16), jnp.float32),
            pltpu.VMEM_SHARED((NPAD, 16), jnp.float32),
            pltpu.VMEM_SHARED((NPAD, 16), jnp.float32),
        ],
    )
    def deg_kernel(src_hbm, dst_hbm, ones_hbm, zeros_hbm, out_hbm,
                   src_v, dst_v, ones_v, acc_out, acc_in):
        cid = lax.axis_index("c")
        sid = lax.axis_index("s")
        wid = sid * NCORE + cid
        r0 = sid * RPS
        pltpu.sync_copy(zeros_hbm, acc_out.at[pl.ds(r0, RPS)])
        pltpu.sync_copy(zeros_hbm, acc_in.at[pl.ds(r0, RPS)])
        pltpu.sync_copy(ones_hbm, ones_v)
        plsc.subcore_barrier()

        def body(j, carry):
            base = wid * EPW + j * CHUNK
            pltpu.sync_copy(src_hbm.at[pl.ds(base, CHUNK)], src_v)
            pltpu.sync_copy(dst_hbm.at[pl.ds(base, CHUNK)], dst_v)
            pltpu.sync_copy(ones_v, acc_out.at[src_v], add=True)
            pltpu.sync_copy(ones_v, acc_in.at[dst_v], add=True)
            return carry

        lax.fori_loop(0, CH_PER_W, body, 0)
        plsc.subcore_barrier()
        pltpu.sync_copy(acc_out.at[pl.ds(r0, RPS)],
                        out_hbm.at[cid, 0, pl.ds(r0, RPS)])
        pltpu.sync_copy(acc_in.at[pl.ds(r0, RPS)],
                        out_hbm.at[cid, 1, pl.ds(r0, RPS)])

    return deg_kernel(srcp, dstp, ones_blk, zeros_blk)


def _sc_msgpass(table, srcp, dstp, zeros_blk, d):
    @functools.partial(
        pl.kernel,
        out_type=jax.ShapeDtypeStruct((NCORE, NPAD, d), jnp.float32),
        mesh=_mesh(),
        compiler_params=pltpu.CompilerParams(use_tc_tiling_on_sc=False),
        scratch_types=[
            pltpu.VMEM((CHUNK,), jnp.int32),
            pltpu.VMEM((CHUNK,), jnp.int32),
            pltpu.VMEM((CHUNK, d), jnp.float32),
            pltpu.VMEM_SHARED((NPAD, d), jnp.float32),
            pltpu.SemaphoreType.DMA,
        ],
    )
    def mp_kernel(table_hbm, src_hbm, dst_hbm, zeros_hbm, out_hbm,
                  src_v, dst_v, rows_v, acc, sem):
        cid = lax.axis_index("c")
        sid = lax.axis_index("s")
        wid = sid * NCORE + cid
        r0 = sid * RPS
        pltpu.sync_copy(zeros_hbm, acc.at[pl.ds(r0, RPS)])
        plsc.subcore_barrier()

        def body(j, carry):
            base = wid * EPW + j * CHUNK
            pltpu.sync_copy(src_hbm.at[pl.ds(base, CHUNK)], src_v)
            pltpu.sync_copy(dst_hbm.at[pl.ds(base, CHUNK)], dst_v)
            pltpu.async_copy(table_hbm.at[src_v], rows_v, sem).wait()
            pltpu.sync_copy(rows_v, acc.at[dst_v], add=True)
            return carry

        lax.fori_loop(0, CH_PER_W, body, 0)
        plsc.subcore_barrier()
        pltpu.sync_copy(acc.at[pl.ds(r0, RPS)],
                        out_hbm.at[cid, pl.ds(r0, RPS)])

    return mp_kernel(table, srcp, dstp, zeros_blk)


def _tc_norms(degp):
    def body(deg_ref, ns_ref, nd_ref):
        dout = jnp.sum(deg_ref[0, 0] + deg_ref[1, 0], axis=1, keepdims=True)
        din = jnp.sum(deg_ref[0, 1] + deg_ref[1, 1], axis=1, keepdims=True)
        dout = dout[:NN]
        din = din[:NN]
        ns_ref[...] = jnp.where(dout > 0, lax.rsqrt(jnp.maximum(dout, 1.0)), 0.0)
        nd_ref[...] = jnp.where(din > 0, lax.rsqrt(jnp.maximum(din, 1.0)), 0.0)

    return pl.pallas_call(
        body,
        out_shape=[
            jax.ShapeDtypeStruct((NN, 1), jnp.float32),
            jax.ShapeDtypeStruct((NN, 1), jnp.float32),
        ],
    )(degp)


def _tc_embed(h2d, label, emb):
    def body(h_ref, lab_ref, emb_ref, x1_ref, x2_ref):
        ids = lax.broadcasted_iota(jnp.int32, (NN, IN_DIM), 1)
        oh = (ids == h_ref[...]).astype(jnp.float32)
        x1 = jnp.dot(oh, emb_ref[...], preferred_element_type=jnp.float32)
        x1_ref[...] = x1
        x2_ref[...] = jnp.concatenate(
            [x1, lab_ref[...], jnp.zeros((NN, D2P - D2), jnp.float32)], axis=1)

    return pl.pallas_call(
        body,
        out_shape=[
            jax.ShapeDtypeStruct((NN, HID), jnp.float32),
            jax.ShapeDtypeStruct((NN, D2P), jnp.float32),
        ],
    )(h2d, label, emb)


def _tc_tables(x1, x2, ns):
    def body(x1_ref, x2_ref, ns_ref, hs1_ref, hs2_ref):
        ns = ns_ref[...]
        hs1_ref[:NN, :] = x1_ref[...] * ns
        hs2_ref[:NN, :] = x2_ref[...] * ns

    return pl.pallas_call(
        body,
        out_shape=[
            jax.ShapeDtypeStruct((NPAD, HID), jnp.float32),
            jax.ShapeDtypeStruct((NPAD, D2P), jnp.float32),
        ],
    )(x1, x2, ns)


def _tc_layer_mm(part, nd, snorm, W, b):
    dout = W.shape[1]

    def body(part_ref, nd_ref, sn_ref, w_ref, b_ref, hg_ref):
        agg = (part_ref[0, :NN, :] + part_ref[1, :NN, :]) * nd_ref[...]
        hg = jnp.dot(agg, w_ref[...], preferred_element_type=jnp.float32)
        hg_ref[...] = (hg + b_ref[...]) * sn_ref[...]

    return pl.pallas_call(
        body,
        out_shape=jax.ShapeDtypeStruct((NN, dout), jnp.float32),
    )(part, nd, snorm, W, b)


def _tc_layer_bn(hg, x, ns, gamma, beta, residual, emit_hs):
    dout = hg.shape[1]

    def body(hg_ref, x_ref, ns_ref, g_ref, bt_ref, xo_ref, *maybe_hs):
        hg = hg_ref[...]
        mean = jnp.mean(hg, axis=0, keepdims=True)
        var = jnp.mean((hg - mean) * (hg - mean), axis=0, keepdims=True)
        hg = (hg - mean) * lax.rsqrt(var + 1e-5) * g_ref[...] + bt_ref[...]
        hg = jnp.maximum(hg, 0.0)
        if residual:
            hg = x_ref[...] + hg
        xo_ref[...] = hg
        if emit_hs:
            maybe_hs[0][:NN, :] = hg * ns_ref[...]

    out_shape = [jax.ShapeDtypeStruct((NN, dout), jnp.float32)]
    if emit_hs:
        out_shape.append(jax.ShapeDtypeStruct((NPAD, dout), jnp.float32))
    return pl.pallas_call(body, out_shape=out_shape)(hg, x, ns, gamma, beta)


def _tc_layer(part, x, ns, nd, snorm, W, b, gamma, beta, residual, emit_hs):
    hg = _tc_layer_mm(part, nd, snorm, W, b)
    return _tc_layer_bn(hg, x, ns, gamma, beta, residual, emit_hs)


def _tc_final(x1, x2, label, delta2d, m1, m2):
    def body(x1_ref, x2_ref, lab_ref, d_ref,
             w10, b10, w11, b11, w12, b12,
             w20, b20, w21, b21, w22, b22,
             p_ref, g_ref):
        def mlp(x, wbs):
            for i, (w, b) in enumerate(wbs):
                x = jnp.dot(x, w[...], preferred_element_type=jnp.float32)
                x = x + b[...]
                if i < 2:
                    x = jnp.maximum(x, 0.0)
            return x

        p_ref[...] = mlp(x1_ref[...], [(w10, b10), (w11, b11), (w12, b12)])
        w = mlp(x2_ref[...], [(w20, b20), (w21, b21), (w22, b22)])
        w = jnp.broadcast_to(w, (NN, NC_CLS))
        dv = d_ref[...]
        w = jnp.clip(w, -dv, dv)
        g = (1.0 - w) * lab_ref[...] + w * (1.0 / NC_CLS)
        mean = jnp.mean(g, axis=1, keepdims=True)
        c = g - mean
        std = jnp.sqrt(jnp.sum(c * c, axis=1, keepdims=True) / (NC_CLS - 1))
        g_ref[...] = c / std

    (w10, b10), (w11, b11), (w12, b12) = m1
    (w20, b20), (w21, b21), (w22, b22) = m2
    return pl.pallas_call(
        body,
        out_shape=[
            jax.ShapeDtypeStruct((NN, NC_CLS), jnp.float32),
            jax.ShapeDtypeStruct((NN, NC_CLS), jnp.float32),
        ],
    )(x1, x2, label, delta2d,
      w10, b10.reshape(1, -1), w11, b11.reshape(1, -1), w12, b12.reshape(1, -1),
      w20, b20.reshape(1, -1), w21, b21.reshape(1, -1), w22, b22.reshape(1, -1))


def _pad_stack2(p, dout_pad):
    W = p["W"]
    Wp = jnp.zeros((D2P, dout_pad), jnp.float32).at[:W.shape[0], :W.shape[1]].set(W)
    b = jnp.zeros((dout_pad,), jnp.float32).at[:p["b"].shape[0]].set(p["b"])
    g = jnp.zeros((dout_pad,), jnp.float32).at[:p["gamma"].shape[0]].set(p["gamma"])
    bt = jnp.zeros((dout_pad,), jnp.float32).at[:p["beta"].shape[0]].set(p["beta"])
    return Wp, b, g, bt


def kernel(h, edge_index, snorm_n, label, delta, params):
    src = edge_index[0].astype(jnp.int32)
    dst = edge_index[1].astype(jnp.int32)
    pad = jnp.full((EPAD - EE,), NN, jnp.int32)
    srcp = jnp.concatenate([src, pad])
    dstp = jnp.concatenate([dst, pad])

    ones_blk = jnp.zeros((CHUNK, 16), jnp.float32).at[:, 0].set(1.0)
    zeros16 = jnp.zeros((RPS, 16), jnp.float32)
    zeros128 = jnp.zeros((RPS, HID), jnp.float32)
    zeros144 = jnp.zeros((RPS, D2P), jnp.float32)

    degp = _sc_degrees(srcp, dstp, ones_blk, zeros16)

    h2d = h.astype(jnp.int32).reshape(NN, 1)
    ns, nd = _tc_norms(degp)
    x1, x2 = _tc_embed(h2d, label, params["emb"])
    hs1, hs2 = _tc_tables(x1, x2, ns)

    snorm = snorm_n

    for i, p in enumerate(params["layers1"]):
        part = _sc_msgpass(hs1, srcp, dstp, zeros128, HID)
        outs = _tc_layer(part, x1, ns, nd, snorm, p["W"],
                         p["b"].reshape(1, -1), p["gamma"].reshape(1, -1),
                         p["beta"].reshape(1, -1),
                         residual=True, emit_hs=(i < 3))
        if i < 3:
            x1, hs1 = outs
        else:
            (x1,) = outs

    for i, p in enumerate(params["layers2"]):
        dout_pad = D2P if i < 3 else HID
        Wp, b, g, bt = _pad_stack2(p, dout_pad)
        part = _sc_msgpass(hs2, srcp, dstp, zeros144, D2P)
        outs = _tc_layer(part, x2, ns, nd, snorm, Wp,
                         b.reshape(1, -1), g.reshape(1, -1), bt.reshape(1, -1),
                         residual=(i < 3), emit_hs=(i < 3))
        if i < 3:
            x2, hs2 = outs
        else:
            (x2,) = outs

    p_out, g_hat = _tc_final(x1, x2, label, delta.reshape(1, 1),
                             params["mlp1"], params["mlp2"])
    return p_out, g_hat
